# baseline (speedup 1.0000x reference)
import jax
import jax.numpy as jnp
from jax import lax
from jax.experimental import pallas as pl
from jax.experimental.pallas import tpu as pltpu
from jax.experimental.pallas import tpu_sc as plsc

N = 10000
E = 320000
D = 128
H = 128
NP = 10240
NC, NS = 2, 16
NW = NC * NS
GPT = 80
EP = NW * GPT * 128
RPS = NP // NS
BLK = 512
NBUF = 2
CMASK = (1 << 14) - 1

_MESH = dict(core_axis_name="c", subcore_axis_name="s",
             num_cores=NC, num_subcores=NS)


def _deg_body(ep_hbm, out_hbm, pb, colb, vb, stage, dsem, deg_sh):
    c = lax.axis_index("c")
    s = lax.axis_index("s")
    gid = c * NS + s
    pltpu.sync_copy(ep_hbm.at[pl.ds(gid * GPT, GPT)], pb)
    for i in range(RPS // 16):
        stage[pl.ds(i * 16, 16)] = jnp.zeros((16,), jnp.float32)
    pltpu.sync_copy(stage, deg_sh.at[pl.ds(s * RPS, RPS)])

    def body(j, carry):
        for i in range(8):
            p = pb[j, pl.ds(i * 16, 16)]
            r = lax.shift_right_logical(p, 14)
            cc = p & CMASK
            colb[j, pl.ds(i * 16, 16)] = cc
            vb[j, pl.ds(i * 16, 16)] = jnp.where(
                r != cc, jnp.full((16,), 1.0, jnp.float32),
                jnp.full((16,), 0.0, jnp.float32))
        return carry

    lax.fori_loop(0, GPT, body, 0)
    plsc.subcore_barrier()

    def fire_batch(q, sem):
        def fire(t, carry2):
            j = q * 16 + t
            pltpu.async_copy(vb.at[j], deg_sh.at[colb.at[j]], sem, add=True)
            return carry2

        lax.fori_loop(0, 16, fire, 0)

    def drain(sem):
        pltpu.make_async_copy(
            ep_hbm.at[pl.ds(0, 16)], colb.at[pl.ds(0, 16)], sem).wait()

    for q in range(GPT // 16):
        if q >= 2:
            drain(dsem.at[q % 2])
        fire_batch(q, dsem.at[q % 2])
    drain(dsem.at[(GPT // 16 - 2) % 2])
    drain(dsem.at[(GPT // 16 - 1) % 2])
    plsc.subcore_barrier()
    pltpu.sync_copy(deg_sh.at[pl.ds(s * RPS, RPS)], stage)
    pltpu.sync_copy(stage, out_hbm.at[c, 0, pl.ds(s * RPS, RPS)])


GPS = 40
NST = GPT // GPS


def _agg_body(ep_hbm, g_hbm, out_hbm, pb, rowb, colb, gb0, gb1,
              gsem, ssem, s_sh):
    gbl = (gb0, gb1)
    c = lax.axis_index("c")
    s = lax.axis_index("s")
    gid = c * NS + s

    def zero_row(i, carry):
        for l in range(8):
            gb0[i, pl.ds(l * 16, 16)] = jnp.zeros((16,), jnp.float32)
        return carry

    lax.fori_loop(0, 128, zero_row, 0)
    for k in range(RPS // 128):
        pltpu.async_copy(gb0, s_sh.at[pl.ds(s * RPS + k * 128, 128)], ssem)

    def unpack(j, carry):
        for i in range(8):
            garb = N + i * 16 + lax.iota(jnp.int32, 16)
            p = pb[j, pl.ds(i * 16, 16)]
            r = lax.shift_right_logical(p, 14)
            cc = p & CMASK
            rowb[j, pl.ds(i * 16, 16)] = r
            colb[j, pl.ds(i * 16, 16)] = jnp.where(r == cc, garb, cc)
        return carry

    def ring():
        pltpu.async_copy(g_hbm.at[rowb.at[0]], gb0, gsem.at[0])

        def outer(o, carry):
            for b in range(NBUF):
                j = o * NBUF + b
                bn = (b + 1) % NBUF
                pltpu.make_async_copy(
                    g_hbm.at[pl.ds(0, 128)], gbl[b], gsem.at[b]).wait()
                pltpu.async_copy(
                    g_hbm.at[rowb.at[j + 1]], gbl[bn], gsem.at[bn])
                pltpu.sync_copy(gbl[b], s_sh.at[colb.at[j]], add=True)
            return carry

        lax.fori_loop(0, GPS // NBUF - 1, outer, 0)
        for j in (GPS - 2, GPS - 1):
            b = j % NBUF
            bn = (b + 1) % NBUF
            pltpu.make_async_copy(
                g_hbm.at[pl.ds(0, 128)], gbl[b], gsem.at[b]).wait()
            if j + 1 < GPS:
                pltpu.async_copy(
                    g_hbm.at[rowb.at[j + 1]], gbl[bn], gsem.at[bn])
            pltpu.sync_copy(gbl[b], s_sh.at[colb.at[j]], add=True)

    pltpu.sync_copy(ep_hbm.at[pl.ds(gid * GPT, GPS)], pb)
    lax.fori_loop(0, GPS, unpack, 0)
    for k in range(RPS // 128):
        pltpu.make_async_copy(g_hbm.at[pl.ds(0, 128)], gb0, ssem).wait()
    plsc.subcore_barrier()
    ring()
    for st in range(1, NST):
        pltpu.sync_copy(ep_hbm.at[pl.ds(gid * GPT + st * GPS, GPS)], pb)
        lax.fori_loop(0, GPS, unpack, 0)
        ring()

    plsc.subcore_barrier()
    NRO = RPS // 128
    for k in range(NRO):
        bb = k % 2
        if k >= 2:
            pltpu.make_async_copy(
                g_hbm.at[pl.ds(0, 128)], gbl[bb], gsem.at[bb]).wait()
        pltpu.sync_copy(s_sh.at[pl.ds(s * RPS + k * 128, 128)], gbl[bb])
        pltpu.async_copy(
            gbl[bb], out_hbm.at[c, pl.ds(s * RPS + k * 128, 128)], gsem.at[bb])
    for k in (NRO - 2, NRO - 1):
        pltpu.make_async_copy(
            g_hbm.at[pl.ds(0, 128)], gbl[k % 2], gsem.at[k % 2]).wait()


_deg_call = pl.kernel(
    _deg_body,
    out_type=jax.ShapeDtypeStruct((NC, 1, NP), jnp.float32),
    mesh=plsc.VectorSubcoreMesh(**_MESH),
    scratch_types=[
        pltpu.VMEM((GPT, 128), jnp.int32),
        pltpu.VMEM((GPT, 128), jnp.int32),
        pltpu.VMEM((GPT, 128), jnp.float32),
        pltpu.VMEM((RPS,), jnp.float32),
        pltpu.SemaphoreType.DMA((2,)),
        pltpu.VMEM_SHARED((NP,), jnp.float32),
    ],
)

_agg_call = pl.kernel(
    _agg_body,
    out_type=jax.ShapeDtypeStruct((NC, NP, D), jnp.float32),
    mesh=plsc.VectorSubcoreMesh(**_MESH),
    scratch_types=[
        pltpu.VMEM((GPS, 128), jnp.int32),
        pltpu.VMEM((GPS, 128), jnp.int32),
        pltpu.VMEM((GPS, 128), jnp.int32),
        pltpu.VMEM((128, D), jnp.float32),
        pltpu.VMEM((128, D), jnp.float32),
        pltpu.SemaphoreType.DMA((NBUF,)),
        pltpu.SemaphoreType.DMA,
        pltpu.VMEM_SHARED((NP, D), jnp.float32),
    ],
)


EBLK = 320


def _pack_body(eb, ep_out):
    i = pl.program_id(0)
    rows = eb[0]
    cols = eb[1]
    gr = i * EBLK + lax.broadcasted_iota(jnp.int32, (EBLK, 128), 0)
    flat = gr * 128 + lax.broadcasted_iota(jnp.int32, (EBLK, 128), 1)
    padv = (flat & 8191) * ((1 << 14) + 1)
    ep_out[...] = jnp.where(flat < E, (rows << 14) | cols, padv)


_pack_call = pl.pallas_call(
    _pack_body,
    out_shape=jax.ShapeDtypeStruct((EP // 128, 128), jnp.int32),
    grid=(EP // 128 // EBLK,),
    in_specs=[pl.BlockSpec((2, EBLK, 128), lambda i: (0, i, 0))],
    out_specs=pl.BlockSpec((EBLK, 128), lambda i: (i, 0)),
)


def _dense_body(xb, wb, dtb, g_out, dis_out):
    dt = dtb[...][:, 0, :]
    deg = dt[0:1, :] + dt[1:2, :] + 1.0
    dis = jnp.transpose(lax.rsqrt(deg), (1, 0))
    h = lax.dot_general(xb[...], wb[...], (((1,), (1,)), ((), ())),
                        preferred_element_type=jnp.float32)
    g_out[...] = h * dis
    dis_out[...] = jnp.broadcast_to(dis, (BLK, 8))


_dense_call = pl.pallas_call(
    _dense_body,
    out_shape=(jax.ShapeDtypeStruct((NP, D), jnp.float32),
               jax.ShapeDtypeStruct((NP, 8), jnp.float32)),
    grid=(NP // BLK,),
    in_specs=[
        pl.BlockSpec((BLK, D), lambda i: (i, 0)),
        pl.BlockSpec((D, D), lambda i: (0, 0)),
        pl.BlockSpec((2, 1, BLK), lambda i: (0, 0, i)),
    ],
    out_specs=(pl.BlockSpec((BLK, D), lambda i: (i, 0)),
               pl.BlockSpec((BLK, 8), lambda i: (i, 0))),
)

FBLK = 1000


def _final_body(sb, gbl, disb, bb, ob):
    dis = disb[:, 0:1]
    ob[...] = dis * (sb[0] + sb[1] + gbl[...]) + bb[...]


_final_call = pl.pallas_call(
    _final_body,
    out_shape=jax.ShapeDtypeStruct((N, H), jnp.float32),
    grid=(N // FBLK,),
    in_specs=[
        pl.BlockSpec((2, FBLK, H), lambda i: (0, i, 0)),
        pl.BlockSpec((FBLK, H), lambda i: (i, 0)),
        pl.BlockSpec((FBLK, 8), lambda i: (i, 0)),
        pl.BlockSpec((1, H), lambda i: (0, 0)),
    ],
    out_specs=pl.BlockSpec((FBLK, H), lambda i: (i, 0)),
)


def kernel(x, edge_index, W, bias):
    ep = _pack_call(edge_index.reshape(2, E // 128, 128))
    xp = jnp.pad(x, ((0, NP - N), (0, 0)))

    deg_p = _deg_call(ep)
    g, dis8 = _dense_call(xp, W, deg_p)
    S = _agg_call(ep, g)
    return _final_call(S, g, dis8, bias.reshape(1, H))
